# Initial kernel scaffold; baseline (speedup 1.0000x reference)
import jax
import jax.numpy as jnp
from jax.experimental import pallas as pl


def kernel(outputs, batch, is_global):
    raise NotImplementedError("write your pallas kernel here")


# speedup vs baseline: 6.8052x; 6.8052x over previous
import functools

import jax
import jax.numpy as jnp
from jax import lax
from jax.experimental import pallas as pl
from jax.experimental.pallas import tpu as pltpu
from jax.experimental.pallas import tpu_sc as plsc

_NUM_SEGMENTS = 1024
_N = 320000
_D = 128

_NC = 2
_NS = 16
_L = 16
_NW = _NC * _NS
_CH = _N // _NW
_CHV = _CH // _L
_HIST = _L * _NUM_SEGMENTS

_COL_BLK = 8000


def _col_body(x_ref, o_ref):
    o_ref[...] = x_ref[:, 0:1]


def _extract_col0(outputs):
    col = pl.pallas_call(
        _col_body,
        grid=(_N // _COL_BLK,),
        in_specs=[pl.BlockSpec((_COL_BLK, _D), lambda i: (i, 0))],
        out_specs=pl.BlockSpec((_COL_BLK, 1), lambda i: (i, 0)),
        out_shape=jax.ShapeDtypeStruct((_N, 1), jnp.float32),
    )(outputs)
    return col.reshape(_N)


def _sc_partials(col0, batch_i32):
    mesh = plsc.VectorSubcoreMesh(core_axis_name="c", subcore_axis_name="s")

    @functools.partial(
        pl.kernel,
        out_type=(
            jax.ShapeDtypeStruct((_NW, _HIST), jnp.float32),
            jax.ShapeDtypeStruct((_NW, _HIST), jnp.float32),
        ),
        mesh=mesh,
        compiler_params=pltpu.CompilerParams(needs_layout_passes=False),
        scratch_types=[
            pltpu.VMEM((_CH,), jnp.int32),
            pltpu.VMEM((_CH,), jnp.float32),
            pltpu.VMEM((_HIST,), jnp.float32),
            pltpu.VMEM((_HIST,), jnp.float32),
        ],
    )
    def k(col_hbm, ids_hbm, parts_s, parts_c, ids_v, vals_v, acc_s, acc_c):
        wid = lax.axis_index("s") * _NC + lax.axis_index("c")
        base = wid * _CH

        pltpu.sync_copy(ids_hbm.at[pl.ds(base, _CH)], ids_v)
        pltpu.sync_copy(col_hbm.at[pl.ds(base, _CH)], vals_v)

        zeros = jnp.zeros((_L,), jnp.float32)

        def zero_body(i, _):
            acc_s[pl.ds(i * _L, _L)] = zeros
            acc_c[pl.ds(i * _L, _L)] = zeros
            return None

        lax.fori_loop(0, _HIST // _L, zero_body, None, unroll=4)

        lane_off = lax.iota(jnp.int32, _L) * _NUM_SEGMENTS
        ones = jnp.ones((_L,), jnp.float32)

        def acc_body(t, _):
            ids = ids_v[pl.ds(t * _L, _L)]
            vals = vals_v[pl.ds(t * _L, _L)]
            addr = ids + lane_off
            plsc.addupdate_scatter(acc_s, [addr], vals)
            plsc.addupdate_scatter(acc_c, [addr], ones)
            return None

        lax.fori_loop(0, _CHV, acc_body, None, unroll=4)

        pltpu.sync_copy(acc_s, parts_s.at[wid])
        pltpu.sync_copy(acc_c, parts_c.at[wid])

    return k(col0, batch_i32)


def _finish_body(ps_ref, pc_ref, o_ref):
    s = jnp.sum(ps_ref[...], axis=0)
    c = jnp.sum(pc_ref[...], axis=0)
    o_ref[...] = (s / jnp.maximum(c, 1.0))[None, :]


def kernel(outputs, batch, is_global):
    del is_global
    batch_i32 = batch.astype(jnp.int32)
    col0 = _extract_col0(outputs)
    parts_s, parts_c = _sc_partials(col0, batch_i32)
    ps = parts_s.reshape(_NW * _L, _NUM_SEGMENTS)
    pc = parts_c.reshape(_NW * _L, _NUM_SEGMENTS)
    score2d = pl.pallas_call(
        _finish_body,
        out_shape=jax.ShapeDtypeStruct((1, _NUM_SEGMENTS), jnp.float32),
    )(ps, pc)
    return score2d.reshape(_NUM_SEGMENTS)


# speedup vs baseline: 13.8482x; 2.0349x over previous
import functools

import jax
import jax.numpy as jnp
from jax import lax
from jax.experimental import pallas as pl
from jax.experimental.pallas import tpu as pltpu
from jax.experimental.pallas import tpu_sc as plsc

_NUM_SEGMENTS = 1024
_N = 320000
_D = 128

_NC = 2
_NS = 16
_L = 16
_NW = _NC * _NS
_CH = _N // _NW
_CHV = _CH // _L
_HIST = _L * _NUM_SEGMENTS


def _sc_partials(outputs, batch_i32):
    mesh = plsc.VectorSubcoreMesh(core_axis_name="c", subcore_axis_name="s")

    @functools.partial(
        pl.kernel,
        out_type=(
            jax.ShapeDtypeStruct((_NW * _HIST,), jnp.float32),
            jax.ShapeDtypeStruct((_NW * _HIST,), jnp.float32),
        ),
        mesh=mesh,
        compiler_params=pltpu.CompilerParams(
            use_tc_tiling_on_sc=False, needs_layout_passes=False
        ),
        scratch_types=[
            pltpu.VMEM((_CH,), jnp.int32),
            pltpu.VMEM((_CH, 1), jnp.float32),
            pltpu.VMEM((_HIST,), jnp.float32),
            pltpu.VMEM((_HIST,), jnp.float32),
        ],
    )
    def k(out_hbm, ids_hbm, parts_s, parts_c, ids_v, vals_v, acc_s, acc_c):
        wid = lax.axis_index("s") * _NC + lax.axis_index("c")
        base = wid * _CH

        pltpu.sync_copy(ids_hbm.at[pl.ds(base, _CH)], ids_v)
        pltpu.sync_copy(out_hbm.at[pl.ds(base, _CH), pl.ds(0, 1)], vals_v)

        zeros = jnp.zeros((_L,), jnp.float32)

        def zero_body(i, _):
            acc_s[pl.ds(i * _L, _L)] = zeros
            acc_c[pl.ds(i * _L, _L)] = zeros
            return None

        lax.fori_loop(0, _HIST // _L, zero_body, None, unroll=4)

        lane_iota = lax.iota(jnp.int32, _L)
        lane_off = lane_iota * _NUM_SEGMENTS
        col_zero = jnp.zeros((_L,), jnp.int32)
        ones = jnp.ones((_L,), jnp.float32)

        def acc_body(t, _):
            ids = ids_v[pl.ds(t * _L, _L)]
            vals = plsc.load_gather(vals_v, [t * _L + lane_iota, col_zero])
            addr = ids + lane_off
            plsc.addupdate_scatter(acc_s, [addr], vals)
            plsc.addupdate_scatter(acc_c, [addr], ones)
            return None

        lax.fori_loop(0, _CHV, acc_body, None, unroll=4)

        pltpu.sync_copy(acc_s, parts_s.at[pl.ds(wid * _HIST, _HIST)])
        pltpu.sync_copy(acc_c, parts_c.at[pl.ds(wid * _HIST, _HIST)])

    return k(outputs, batch_i32)


def _finish_body(ps_ref, pc_ref, o_ref):
    s = jnp.sum(ps_ref[...], axis=0)
    c = jnp.sum(pc_ref[...], axis=0)
    o_ref[...] = (s / jnp.maximum(c, 1.0))[None, :]


def kernel(outputs, batch, is_global):
    del is_global
    batch_i32 = batch.astype(jnp.int32)
    parts_s, parts_c = _sc_partials(outputs, batch_i32)
    ps = parts_s.reshape(_NW * _L, _NUM_SEGMENTS)
    pc = parts_c.reshape(_NW * _L, _NUM_SEGMENTS)
    score2d = pl.pallas_call(
        _finish_body,
        out_shape=jax.ShapeDtypeStruct((1, _NUM_SEGMENTS), jnp.float32),
    )(ps, pc)
    return score2d.reshape(_NUM_SEGMENTS)


# speedup vs baseline: 13.9461x; 1.0071x over previous
import functools

import jax
import jax.numpy as jnp
from jax import lax
from jax.experimental import pallas as pl
from jax.experimental.pallas import tpu as pltpu
from jax.experimental.pallas import tpu_sc as plsc

_NUM_SEGMENTS = 1024
_N = 320000
_D = 128

_NC = 2
_NS = 16
_L = 16
_NW = _NC * _NS
_CH = _N // _NW
_CHV = _CH // _L
_HIST = _L * _NUM_SEGMENTS


def _sc_partials(outputs, batch_i32):
    mesh = plsc.VectorSubcoreMesh(core_axis_name="c", subcore_axis_name="s")

    @functools.partial(
        pl.kernel,
        out_type=(
            jax.ShapeDtypeStruct((_NW * _HIST,), jnp.float32),
            jax.ShapeDtypeStruct((_NW * _HIST,), jnp.float32),
        ),
        mesh=mesh,
        compiler_params=pltpu.CompilerParams(
            use_tc_tiling_on_sc=False, needs_layout_passes=False
        ),
        scratch_types=[
            pltpu.VMEM((_CH,), jnp.int32),
            pltpu.VMEM((_CH, 1), jnp.float32),
            pltpu.VMEM((_HIST,), jnp.float32),
            pltpu.VMEM((_HIST,), jnp.float32),
            pltpu.SemaphoreType.DMA,
        ],
    )
    def k(out_hbm, ids_hbm, parts_s, parts_c, ids_v, vals_v, acc_s, acc_c, sem):
        wid = lax.axis_index("s") * _NC + lax.axis_index("c")
        base = wid * _CH

        nstream = 8
        sub = _CH // nstream
        cps = [pltpu.async_copy(ids_hbm.at[pl.ds(base, _CH)], ids_v, sem)]
        for j in range(nstream):
            cps.append(pltpu.async_copy(
                out_hbm.at[pl.ds(base + j * sub, sub), pl.ds(0, 1)],
                vals_v.at[pl.ds(j * sub, sub), pl.ds(0, 1)],
                sem,
            ))
        for cp in cps:
            cp.wait()

        zeros = jnp.zeros((_L,), jnp.float32)

        def zero_body(i, _):
            acc_s[pl.ds(i * _L, _L)] = zeros
            acc_c[pl.ds(i * _L, _L)] = zeros
            return None

        lax.fori_loop(0, _HIST // _L, zero_body, None, unroll=4)

        lane_iota = lax.iota(jnp.int32, _L)
        lane_off = lane_iota * _NUM_SEGMENTS
        col_zero = jnp.zeros((_L,), jnp.int32)
        ones = jnp.ones((_L,), jnp.float32)

        def acc_body(t, _):
            ids = ids_v[pl.ds(t * _L, _L)]
            vals = plsc.load_gather(vals_v, [t * _L + lane_iota, col_zero])
            addr = ids + lane_off
            plsc.addupdate_scatter(acc_s, [addr], vals)
            plsc.addupdate_scatter(acc_c, [addr], ones)
            return None

        lax.fori_loop(0, _CHV, acc_body, None, unroll=4)

        pltpu.sync_copy(acc_s, parts_s.at[pl.ds(wid * _HIST, _HIST)])
        pltpu.sync_copy(acc_c, parts_c.at[pl.ds(wid * _HIST, _HIST)])

    return k(outputs, batch_i32)


def _finish_body(ps_ref, pc_ref, o_ref):
    s = jnp.sum(ps_ref[...], axis=0)
    c = jnp.sum(pc_ref[...], axis=0)
    o_ref[...] = (s / jnp.maximum(c, 1.0))[None, :]


def kernel(outputs, batch, is_global):
    del is_global
    batch_i32 = batch.astype(jnp.int32)
    parts_s, parts_c = _sc_partials(outputs, batch_i32)
    ps = parts_s.reshape(_NW * _L, _NUM_SEGMENTS)
    pc = parts_c.reshape(_NW * _L, _NUM_SEGMENTS)
    score2d = pl.pallas_call(
        _finish_body,
        out_shape=jax.ShapeDtypeStruct((1, _NUM_SEGMENTS), jnp.float32),
    )(ps, pc)
    return score2d.reshape(_NUM_SEGMENTS)


# speedup vs baseline: 23.0356x; 1.6518x over previous
import functools

import jax
import jax.numpy as jnp
from jax import lax
from jax.experimental import pallas as pl
from jax.experimental.pallas import tpu as pltpu
from jax.experimental.pallas import tpu_sc as plsc

_NUM_SEGMENTS = 1024
_N = 320000
_D = 128

_NC = 2
_NS = 16
_L = 16
_NW = _NC * _NS
_CH = _N // _NW
_CHV = _CH // _L
_HIST = _L * _NUM_SEGMENTS

_GW = 128
_GROWS = 80
_GFIRE = 8


def _sc_partials(out_flat, batch_i32):
    mesh = plsc.VectorSubcoreMesh(core_axis_name="c", subcore_axis_name="s")

    @functools.partial(
        pl.kernel,
        out_type=(
            jax.ShapeDtypeStruct((_NW * _HIST,), jnp.float32),
            jax.ShapeDtypeStruct((_NW * _HIST,), jnp.float32),
        ),
        mesh=mesh,
        compiler_params=pltpu.CompilerParams(
            use_tc_tiling_on_sc=False, needs_layout_passes=False
        ),
        scratch_types=[
            pltpu.VMEM((_CH,), jnp.int32),
            pltpu.VMEM((_GROWS, _GW), jnp.int32),
            pltpu.VMEM((_GROWS, _GW), jnp.float32),
            pltpu.VMEM((_HIST,), jnp.float32),
            pltpu.VMEM((_HIST,), jnp.float32),
            pltpu.SemaphoreType.DMA,
        ],
    )
    def k(flat_hbm, ids_hbm, parts_s, parts_c,
          ids_v, idx_v, vals_g, acc_s, acc_c, sem):
        wid = lax.axis_index("s") * _NC + lax.axis_index("c")
        base = wid * _CH

        ids_cp = pltpu.async_copy(ids_hbm.at[pl.ds(base, _CH)], ids_v, sem)

        lane_iota = lax.iota(jnp.int32, _L)
        last = base + (_CH - 1)

        def fill_body(j, _):
            for kk in range(_GW // _L):
                elem = base + j * _GW + kk * _L + lane_iota
                elem = jnp.minimum(elem, last)
                idx_v[j, pl.ds(kk * _L, _L)] = elem * _D
            return None

        lax.fori_loop(0, _GROWS, fill_body, None)

        zeros = jnp.zeros((_L,), jnp.float32)

        def zero_body(i, _):
            acc_s[pl.ds(i * _L, _L)] = zeros
            acc_c[pl.ds(i * _L, _L)] = zeros
            return None

        lax.fori_loop(0, _HIST // _L, zero_body, None, unroll=4)

        def gather_body(g, _):
            row = g * _GFIRE
            cps = []
            for r in range(_GFIRE):
                cps.append(pltpu.async_copy(
                    flat_hbm.at[idx_v.at[row + r]], vals_g.at[row + r], sem))
            for cp in cps:
                cp.wait()
            return None

        lax.fori_loop(0, _GROWS // _GFIRE, gather_body, None)

        ids_cp.wait()

        lane_off = lane_iota * _NUM_SEGMENTS
        ones = jnp.ones((_L,), jnp.float32)

        def acc_body(t, _):
            ids = ids_v[pl.ds(t * _L, _L)]
            vals = vals_g[t // 8, pl.ds((t % 8) * _L, _L)]
            addr = ids + lane_off
            plsc.addupdate_scatter(acc_s, [addr], vals)
            plsc.addupdate_scatter(acc_c, [addr], ones)
            return None

        lax.fori_loop(0, _CHV, acc_body, None, unroll=4)

        pltpu.sync_copy(acc_s, parts_s.at[pl.ds(wid * _HIST, _HIST)])
        pltpu.sync_copy(acc_c, parts_c.at[pl.ds(wid * _HIST, _HIST)])

    return k(out_flat, batch_i32)


def _finish_body(ps_ref, pc_ref, o_ref):
    s = jnp.sum(ps_ref[...], axis=0)
    c = jnp.sum(pc_ref[...], axis=0)
    o_ref[...] = (s / jnp.maximum(c, 1.0))[None, :]


def kernel(outputs, batch, is_global):
    del is_global
    batch_i32 = batch.astype(jnp.int32)
    parts_s, parts_c = _sc_partials(outputs.reshape(_N * _D), batch_i32)
    ps = parts_s.reshape(_NW * _L, _NUM_SEGMENTS)
    pc = parts_c.reshape(_NW * _L, _NUM_SEGMENTS)
    score2d = pl.pallas_call(
        _finish_body,
        out_shape=jax.ShapeDtypeStruct((1, _NUM_SEGMENTS), jnp.float32),
    )(ps, pc)
    return score2d.reshape(_NUM_SEGMENTS)


# speedup vs baseline: 26.4946x; 1.1502x over previous
import functools

import jax
import jax.numpy as jnp
from jax import lax
from jax.experimental import pallas as pl
from jax.experimental.pallas import tpu as pltpu
from jax.experimental.pallas import tpu_sc as plsc

_NUM_SEGMENTS = 1024
_N = 320000
_D = 128

_NC = 2
_NS = 16
_L = 16
_NW = _NC * _NS
_CH = _N // _NW
_CHV = _CH // _L
_HIST = _L * _NUM_SEGMENTS

_GW = 128
_GROWS = 80
_GFIRE = 8


def _sc_partials(out_flat, batch_i32):
    mesh = plsc.VectorSubcoreMesh(core_axis_name="c", subcore_axis_name="s")

    @functools.partial(
        pl.kernel,
        out_type=(
            jax.ShapeDtypeStruct((_NW * _HIST,), jnp.float32),
            jax.ShapeDtypeStruct((_NW * _HIST,), jnp.float32),
        ),
        mesh=mesh,
        compiler_params=pltpu.CompilerParams(
            use_tc_tiling_on_sc=False, needs_layout_passes=False
        ),
        scratch_types=[
            pltpu.VMEM((_CH,), jnp.int32),
            pltpu.VMEM((_GROWS, _GW), jnp.int32),
            pltpu.VMEM((_GROWS, _GW), jnp.float32),
            pltpu.VMEM((_HIST,), jnp.float32),
            pltpu.VMEM((_HIST,), jnp.float32),
            pltpu.SemaphoreType.DMA,
            pltpu.SemaphoreType.DMA,
            pltpu.SemaphoreType.DMA,
        ],
    )
    def k(flat_hbm, ids_hbm, parts_s, parts_c,
          ids_v, idx_v, vals_g, acc_s, acc_c, sem_i, sem_a, sem_b):
        wid = lax.axis_index("s") * _NC + lax.axis_index("c")
        base = wid * _CH

        ids_cp = pltpu.async_copy(ids_hbm.at[pl.ds(base, _CH)], ids_v, sem_i)

        lane_iota = lax.iota(jnp.int32, _L)
        last = base + (_CH - 1)
        sems = (sem_a, sem_b)
        ngroups = _GROWS // _GFIRE

        def fill_group(g):
            for r in range(_GFIRE):
                j = g * _GFIRE + r
                for kk in range(_GW // _L):
                    elem = base + j * _GW + kk * _L + lane_iota
                    elem = jnp.minimum(elem, last)
                    idx_v[j, pl.ds(kk * _L, _L)] = elem * _D

        def fire_group(g):
            s = sems[g % 2]
            return [
                pltpu.async_copy(
                    flat_hbm.at[idx_v.at[g * _GFIRE + r]],
                    vals_g.at[g * _GFIRE + r], s)
                for r in range(_GFIRE)
            ]

        fill_group(0)
        inflight = {0: fire_group(0)}

        zeros = jnp.zeros((_L,), jnp.float32)

        def zero_body(i, _):
            acc_s[pl.ds(i * _L, _L)] = zeros
            acc_c[pl.ds(i * _L, _L)] = zeros
            return None

        lax.fori_loop(0, _HIST // _L, zero_body, None, unroll=4)

        ids_cp.wait()

        lane_off = lane_iota * _NUM_SEGMENTS
        ones = jnp.ones((_L,), jnp.float32)

        def acc_body(t, _):
            ids = ids_v[pl.ds(t * _L, _L)]
            vals = vals_g[t // 8, pl.ds((t % 8) * _L, _L)]
            addr = ids + lane_off
            plsc.addupdate_scatter(acc_s, [addr], vals)
            plsc.addupdate_scatter(acc_c, [addr], ones)
            return None

        vpg = _GFIRE * _GW // _L
        for g in range(ngroups):
            if g + 1 < ngroups:
                fill_group(g + 1)
                inflight[g + 1] = fire_group(g + 1)
            for cp in inflight.pop(g):
                cp.wait()
            lax.fori_loop(g * vpg, min((g + 1) * vpg, _CHV),
                          acc_body, None, unroll=4)

        pltpu.sync_copy(acc_s, parts_s.at[pl.ds(wid * _HIST, _HIST)])
        pltpu.sync_copy(acc_c, parts_c.at[pl.ds(wid * _HIST, _HIST)])

    return k(out_flat, batch_i32)


def _finish_body(ps_ref, pc_ref, o_ref):
    s = jnp.sum(ps_ref[...], axis=0)
    c = jnp.sum(pc_ref[...], axis=0)
    o_ref[...] = (s / jnp.maximum(c, 1.0))[None, :]


def kernel(outputs, batch, is_global):
    del is_global
    batch_i32 = batch.astype(jnp.int32)
    parts_s, parts_c = _sc_partials(outputs.reshape(_N * _D), batch_i32)
    ps = parts_s.reshape(_NW * _L, _NUM_SEGMENTS)
    pc = parts_c.reshape(_NW * _L, _NUM_SEGMENTS)
    score2d = pl.pallas_call(
        _finish_body,
        out_shape=jax.ShapeDtypeStruct((1, _NUM_SEGMENTS), jnp.float32),
    )(ps, pc)
    return score2d.reshape(_NUM_SEGMENTS)
